# baseline (speedup 1.0000x reference)
import functools

import jax
import jax.numpy as jnp
from jax import lax
from jax.experimental import pallas as pl
from jax.experimental.pallas import tpu as pltpu
from jax.experimental.pallas import tpu_sc as plsc

N = 10000
E = 320000
D = 128
OUT = 128
EPS = 1e-5

NSUB = 16
CHUNK = 128
NBUF = 3
NCHUNK = 159
E_PAD = NSUB * NCHUNK * CHUNK
EDGES_PER_SUB = E_PAD // NSUB
ROWS_A = 624
ROWS_LAST = N - 15 * ROWS_A
STREAM_ROWS = 32
ROWDMA = CHUNK - STREAM_ROWS


def _sc_messages(x, x_flat, gidx, sidx, e_pad, zeros):
    mesh = plsc.VectorSubcoreMesh(core_axis_name="c", subcore_axis_name="s")

    @functools.partial(
        pl.kernel,
        mesh=mesh,
        out_type=(
            jax.ShapeDtypeStruct((N, D), jnp.float32),
            jax.ShapeDtypeStruct((N, D), jnp.float32),
        ),
        scratch_types=(
            [pltpu.VMEM_SHARED((N, D), jnp.float32)]
            + [pltpu.VMEM((CHUNK, D), jnp.float32) for _ in range(NBUF)]
            + [pltpu.VMEM((CHUNK,), jnp.int32) for _ in range(NBUF)]
            + [pltpu.VMEM((CHUNK,), jnp.int32) for _ in range(NBUF)]
            + [pltpu.VMEM((CHUNK,), jnp.float32) for _ in range(NBUF)]
            + [pltpu.SemaphoreType.DMA for _ in range(3 * NBUF)]
        ),
    )
    def sc_kernel(x_hbm, xf_hbm, gidx_hbm, sidx_hbm, e_hbm, zero_hbm,
                  mi_hbm, mo_hbm, acc, *rest):
        rows = rest[0:NBUF]
        idxg = rest[NBUF:2 * NBUF]
        idxs = rest[2 * NBUF:3 * NBUF]
        ew = rest[3 * NBUF:4 * NBUF]
        gsem = rest[4 * NBUF:5 * NBUF]
        ssem = rest[5 * NBUF:6 * NBUF]
        isem = rest[6 * NBUF:7 * NBUF]
        c = lax.axis_index("c")
        s = lax.axis_index("s")
        row0 = pl.multiple_of(s * ROWS_A, 8)

        def stage_idx(i, b):
            pltpu.async_copy(gidx_hbm.at[c, s, i], idxg[b], isem[b])
            pltpu.async_copy(sidx_hbm.at[c, s, i], idxs[b], isem[b])
            pltpu.async_copy(e_hbm.at[s, i], ew[b], isem[b])

        def wait_idx(b):
            pltpu.make_async_copy(gidx_hbm.at[c, s, 0], idxg[b], isem[b]).wait()
            pltpu.make_async_copy(sidx_hbm.at[c, s, 0], idxs[b], isem[b]).wait()
            pltpu.make_async_copy(e_hbm.at[s, 0], ew[b], isem[b]).wait()

        def issue_gather(b):
            pltpu.async_copy(x_hbm.at[idxg[b].at[pl.ds(0, STREAM_ROWS)]],
                             rows[b].at[pl.ds(0, STREAM_ROWS)], gsem[b])
            for h in range(ROWDMA // 16):
                iv = idxg[b][pl.ds(STREAM_ROWS + h * 16, 16)]
                for j in range(16):
                    r = STREAM_ROWS + h * 16 + j
                    off = pl.multiple_of(iv[j] * D, 8)
                    pltpu.async_copy(xf_hbm.at[pl.ds(off, D)],
                                     rows[b].at[r], gsem[b])

        for b in range(NBUF):
            stage_idx(b, b)
        for b in range(2):
            wait_idx(b)
            issue_gather(b)

        @pl.when(s < NSUB - 1)
        def _():
            pltpu.sync_copy(zero_hbm.at[pl.ds(0, ROWS_A)],
                            acc.at[pl.ds(row0, ROWS_A)])

        @pl.when(s == NSUB - 1)
        def _():
            pltpu.sync_copy(zero_hbm.at[pl.ds(0, ROWS_LAST)],
                            acc.at[pl.ds(row0, ROWS_LAST)])

        plsc.subcore_barrier()

        def group_body(g, carry):
            i0 = g * NBUF
            for b in range(NBUF):
                i = i0 + b
                pltpu.make_async_copy(x_hbm.at[idxg[b]], rows[b],
                                      gsem[b]).wait()

                def grp_body(gg, rcarry, b=b):
                    r0 = pl.multiple_of(gg * 16, 16)
                    ev = ew[b][pl.ds(r0, 16)]
                    sv = idxs[b][pl.ds(r0, 16)]
                    for j in range(16):
                        sc = ev[j]
                        for d in range(D // 16):
                            sl = pl.ds(d * 16, 16)
                            rows[b][r0 + j, sl] = rows[b][r0 + j, sl] * sc
                    pltpu.async_copy(rows[b].at[pl.ds(r0, 16)], acc.at[sv],
                                     ssem[b], add=True)
                    return rcarry

                lax.fori_loop(0, CHUNK // 16, grp_body, 0)

                @pl.when(i >= 1)
                def _(b=b):
                    pltpu.make_async_copy(rows[(b + 2) % NBUF],
                                          acc.at[idxs[(b + 2) % NBUF]],
                                          ssem[(b + 2) % NBUF]).wait()

                @pl.when(i + 2 < NCHUNK)
                def _(b=b):
                    b2 = (b + 2) % NBUF
                    wait_idx(b2)
                    issue_gather(b2)

                @pl.when(i + 3 < NCHUNK)
                def _(b=b, i=i):
                    stage_idx(i + 3, b)
            return carry

        lax.fori_loop(0, NCHUNK // NBUF, group_body, 0)
        pltpu.make_async_copy(rows[(NCHUNK - 1) % NBUF],
                              acc.at[idxs[(NCHUNK - 1) % NBUF]],
                              ssem[(NCHUNK - 1) % NBUF]).wait()
        plsc.subcore_barrier()

        for cid, dst in ((0, mi_hbm), (1, mo_hbm)):
            @pl.when((c == cid) & (s < NSUB - 1))
            def _(dst=dst):
                pltpu.sync_copy(acc.at[pl.ds(row0, ROWS_A)],
                                dst.at[pl.ds(row0, ROWS_A)])

            @pl.when((c == cid) & (s == NSUB - 1))
            def _(dst=dst):
                pltpu.sync_copy(acc.at[pl.ds(row0, ROWS_LAST)],
                                dst.at[pl.ds(row0, ROWS_LAST)])

    return sc_kernel(x, x_flat, gidx, sidx, e_pad, zeros)


def _ln_relu(h, g, b):
    mu = jnp.mean(h, axis=-1, keepdims=True)
    var = jnp.mean((h - mu) ** 2, axis=-1, keepdims=True)
    h = (h - mu) / jnp.sqrt(var + EPS) * g + b
    return jnp.maximum(h, 0.0)


BLK = 1000


def _mlp(mi, mo, x, Ws, bs, gs, betas):
    grid = (N // BLK,)

    def tc_body(mi_ref, mo_ref, x_ref,
                W0_ref, b0_ref, g0_ref, t0_ref,
                W1_ref, b1_ref, g1_ref, t1_ref,
                W2_ref, b2_ref, g2_ref, t2_ref,
                W3_ref, b3_ref, g3_ref, t3_ref,
                out_ref):
        w0 = W0_ref[...]
        h = (
            jnp.dot(mi_ref[...], w0[0:D, :], preferred_element_type=jnp.float32)
            + jnp.dot(mo_ref[...], w0[D:2 * D, :], preferred_element_type=jnp.float32)
            + jnp.dot(x_ref[...], w0[2 * D:3 * D, :], preferred_element_type=jnp.float32)
            + b0_ref[...]
        )
        h = _ln_relu(h, g0_ref[...], t0_ref[...])
        for W_ref, b_ref, g_ref, t_ref in (
            (W1_ref, b1_ref, g1_ref, t1_ref),
            (W2_ref, b2_ref, g2_ref, t2_ref),
            (W3_ref, b3_ref, g3_ref, t3_ref),
        ):
            h = jnp.dot(h, W_ref[...], preferred_element_type=jnp.float32) + b_ref[...]
            h = _ln_relu(h, g_ref[...], t_ref[...])
        out_ref[...] = h

    row_spec = pl.BlockSpec((BLK, D), lambda i: (i, 0))
    vec_spec = pl.BlockSpec((1, OUT), lambda i: (0, 0))
    in_specs = [row_spec, row_spec, row_spec]
    for Wshape in ((3 * D, OUT), (OUT, OUT), (OUT, OUT), (OUT, OUT)):
        in_specs.append(pl.BlockSpec(Wshape, lambda i: (0, 0)))
        in_specs.extend([vec_spec, vec_spec, vec_spec])

    args = [mi, mo, x]
    for i in range(4):
        args.extend([Ws[i], bs[i].reshape(1, OUT), gs[i].reshape(1, OUT),
                     betas[i].reshape(1, OUT)])

    return pl.pallas_call(
        tc_body,
        grid=grid,
        in_specs=in_specs,
        out_specs=pl.BlockSpec((BLK, OUT), lambda i: (i, 0)),
        out_shape=jax.ShapeDtypeStruct((N, OUT), jnp.float32),
    )(*args)


def kernel(x, e, edge_index, W0, b0, g0, beta0, W1, b1, g1, beta1,
           W2, b2, g2, beta2, W3, b3, g3, beta3):
    start = edge_index[0]
    end = edge_index[1]
    pad = E_PAD - E
    gidx = jnp.stack([jnp.pad(start, (0, pad)), jnp.pad(end, (0, pad))])
    gidx = gidx.reshape(2, NSUB, NCHUNK, CHUNK)
    sidx = jnp.stack([jnp.pad(end, (0, pad)), jnp.pad(start, (0, pad))])
    sidx = sidx.reshape(2, NSUB, NCHUNK, CHUNK)
    e_pad = jnp.pad(e, (0, pad)).reshape(NSUB, NCHUNK, CHUNK)
    zeros = jnp.zeros((ROWS_LAST, D), jnp.float32)
    x_flat = jnp.concatenate([x.reshape(-1), jnp.zeros((128,), jnp.float32)])
    mi, mo = _sc_messages(x, x_flat, gidx, sidx, e_pad, zeros)
    return _mlp(mi, mo, x, (W0, W1, W2, W3), (b0, b1, b2, b3),
                (g0, g1, g2, g3), (beta0, beta1, beta2, beta3))


# speedup vs baseline: 1.0938x; 1.0938x over previous
import functools

import jax
import jax.numpy as jnp
from jax import lax
from jax.experimental import pallas as pl
from jax.experimental.pallas import tpu as pltpu
from jax.experimental.pallas import tpu_sc as plsc

N = 10000
E = 320000
D = 128
OUT = 128
EPS = 1e-5

NSUB = 16
CHUNK = 128
NBUF = 3
NCHUNK = 159
E_PAD = NSUB * NCHUNK * CHUNK
EDGES_PER_SUB = E_PAD // NSUB
ROWS_A = 624
ROWS_LAST = N - 15 * ROWS_A
STREAM_ROWS = 48
ROWDMA = CHUNK - STREAM_ROWS


def _sc_messages(x, x_flat, gidx, sidx, e_pad, zeros):
    mesh = plsc.VectorSubcoreMesh(core_axis_name="c", subcore_axis_name="s")

    @functools.partial(
        pl.kernel,
        mesh=mesh,
        out_type=(
            jax.ShapeDtypeStruct((N, D), jnp.float32),
            jax.ShapeDtypeStruct((N, D), jnp.float32),
        ),
        scratch_types=(
            [pltpu.VMEM_SHARED((N, D), jnp.float32)]
            + [pltpu.VMEM((CHUNK, D), jnp.float32) for _ in range(NBUF)]
            + [pltpu.VMEM((CHUNK,), jnp.int32) for _ in range(NBUF)]
            + [pltpu.VMEM((CHUNK,), jnp.int32) for _ in range(NBUF)]
            + [pltpu.VMEM((CHUNK,), jnp.float32) for _ in range(NBUF)]
            + [pltpu.SemaphoreType.DMA for _ in range(3 * NBUF)]
        ),
    )
    def sc_kernel(x_hbm, xf_hbm, gidx_hbm, sidx_hbm, e_hbm, zero_hbm,
                  mi_hbm, mo_hbm, acc, *rest):
        rows = rest[0:NBUF]
        idxg = rest[NBUF:2 * NBUF]
        idxs = rest[2 * NBUF:3 * NBUF]
        ew = rest[3 * NBUF:4 * NBUF]
        gsem = rest[4 * NBUF:5 * NBUF]
        ssem = rest[5 * NBUF:6 * NBUF]
        isem = rest[6 * NBUF:7 * NBUF]
        c = lax.axis_index("c")
        s = lax.axis_index("s")
        row0 = pl.multiple_of(s * ROWS_A, 8)

        def stage_idx(i, b):
            pltpu.async_copy(gidx_hbm.at[c, s, i], idxg[b], isem[b])
            pltpu.async_copy(sidx_hbm.at[c, s, i], idxs[b], isem[b])
            pltpu.async_copy(e_hbm.at[s, i], ew[b], isem[b])

        def wait_idx(b):
            pltpu.make_async_copy(gidx_hbm.at[c, s, 0], idxg[b], isem[b]).wait()
            pltpu.make_async_copy(sidx_hbm.at[c, s, 0], idxs[b], isem[b]).wait()
            pltpu.make_async_copy(e_hbm.at[s, 0], ew[b], isem[b]).wait()

        def issue_gather(b):
            pltpu.async_copy(x_hbm.at[idxg[b].at[pl.ds(0, STREAM_ROWS)]],
                             rows[b].at[pl.ds(0, STREAM_ROWS)], gsem[b])
            for h in range(ROWDMA // 16):
                iv = idxg[b][pl.ds(STREAM_ROWS + h * 16, 16)]
                for j in range(16):
                    r = STREAM_ROWS + h * 16 + j
                    off = pl.multiple_of(iv[j] * D, 8)
                    pltpu.async_copy(xf_hbm.at[pl.ds(off, D)],
                                     rows[b].at[r], gsem[b])

        for b in range(NBUF):
            stage_idx(b, b)
        for b in range(2):
            wait_idx(b)
            issue_gather(b)

        @pl.when(s < NSUB - 1)
        def _():
            pltpu.sync_copy(zero_hbm.at[pl.ds(0, ROWS_A)],
                            acc.at[pl.ds(row0, ROWS_A)])

        @pl.when(s == NSUB - 1)
        def _():
            pltpu.sync_copy(zero_hbm.at[pl.ds(0, ROWS_LAST)],
                            acc.at[pl.ds(row0, ROWS_LAST)])

        plsc.subcore_barrier()

        def group_body(g, carry):
            i0 = g * NBUF
            for b in range(NBUF):
                i = i0 + b
                pltpu.make_async_copy(x_hbm.at[idxg[b]], rows[b],
                                      gsem[b]).wait()

                def grp_body(gg, rcarry, b=b):
                    r0 = pl.multiple_of(gg * 16, 16)
                    ev = ew[b][pl.ds(r0, 16)]
                    sv = idxs[b][pl.ds(r0, 16)]
                    for j in range(16):
                        sc = ev[j]
                        for d in range(D // 16):
                            sl = pl.ds(d * 16, 16)
                            rows[b][r0 + j, sl] = rows[b][r0 + j, sl] * sc
                    pltpu.async_copy(rows[b].at[pl.ds(r0, 16)], acc.at[sv],
                                     ssem[b], add=True)
                    return rcarry

                lax.fori_loop(0, CHUNK // 16, grp_body, 0)

                @pl.when(i >= 1)
                def _(b=b):
                    pltpu.make_async_copy(rows[(b + 2) % NBUF],
                                          acc.at[idxs[(b + 2) % NBUF]],
                                          ssem[(b + 2) % NBUF]).wait()

                @pl.when(i + 2 < NCHUNK)
                def _(b=b):
                    b2 = (b + 2) % NBUF
                    wait_idx(b2)
                    issue_gather(b2)

                @pl.when(i + 3 < NCHUNK)
                def _(b=b, i=i):
                    stage_idx(i + 3, b)
            return carry

        lax.fori_loop(0, NCHUNK // NBUF, group_body, 0)
        pltpu.make_async_copy(rows[(NCHUNK - 1) % NBUF],
                              acc.at[idxs[(NCHUNK - 1) % NBUF]],
                              ssem[(NCHUNK - 1) % NBUF]).wait()
        plsc.subcore_barrier()

        for cid, dst in ((0, mi_hbm), (1, mo_hbm)):
            @pl.when((c == cid) & (s < NSUB - 1))
            def _(dst=dst):
                pltpu.sync_copy(acc.at[pl.ds(row0, ROWS_A)],
                                dst.at[pl.ds(row0, ROWS_A)])

            @pl.when((c == cid) & (s == NSUB - 1))
            def _(dst=dst):
                pltpu.sync_copy(acc.at[pl.ds(row0, ROWS_LAST)],
                                dst.at[pl.ds(row0, ROWS_LAST)])

    return sc_kernel(x, x_flat, gidx, sidx, e_pad, zeros)


def _ln_relu(h, g, b):
    mu = jnp.mean(h, axis=-1, keepdims=True)
    var = jnp.mean((h - mu) ** 2, axis=-1, keepdims=True)
    h = (h - mu) / jnp.sqrt(var + EPS) * g + b
    return jnp.maximum(h, 0.0)


BLK = 1000


def _mlp(mi, mo, x, Ws, bs, gs, betas):
    grid = (N // BLK,)

    def tc_body(mi_ref, mo_ref, x_ref,
                W0_ref, b0_ref, g0_ref, t0_ref,
                W1_ref, b1_ref, g1_ref, t1_ref,
                W2_ref, b2_ref, g2_ref, t2_ref,
                W3_ref, b3_ref, g3_ref, t3_ref,
                out_ref):
        w0 = W0_ref[...]
        h = (
            jnp.dot(mi_ref[...], w0[0:D, :], preferred_element_type=jnp.float32)
            + jnp.dot(mo_ref[...], w0[D:2 * D, :], preferred_element_type=jnp.float32)
            + jnp.dot(x_ref[...], w0[2 * D:3 * D, :], preferred_element_type=jnp.float32)
            + b0_ref[...]
        )
        h = _ln_relu(h, g0_ref[...], t0_ref[...])
        for W_ref, b_ref, g_ref, t_ref in (
            (W1_ref, b1_ref, g1_ref, t1_ref),
            (W2_ref, b2_ref, g2_ref, t2_ref),
            (W3_ref, b3_ref, g3_ref, t3_ref),
        ):
            h = jnp.dot(h, W_ref[...], preferred_element_type=jnp.float32) + b_ref[...]
            h = _ln_relu(h, g_ref[...], t_ref[...])
        out_ref[...] = h

    row_spec = pl.BlockSpec((BLK, D), lambda i: (i, 0))
    vec_spec = pl.BlockSpec((1, OUT), lambda i: (0, 0))
    in_specs = [row_spec, row_spec, row_spec]
    for Wshape in ((3 * D, OUT), (OUT, OUT), (OUT, OUT), (OUT, OUT)):
        in_specs.append(pl.BlockSpec(Wshape, lambda i: (0, 0)))
        in_specs.extend([vec_spec, vec_spec, vec_spec])

    args = [mi, mo, x]
    for i in range(4):
        args.extend([Ws[i], bs[i].reshape(1, OUT), gs[i].reshape(1, OUT),
                     betas[i].reshape(1, OUT)])

    return pl.pallas_call(
        tc_body,
        grid=grid,
        in_specs=in_specs,
        out_specs=pl.BlockSpec((BLK, OUT), lambda i: (i, 0)),
        out_shape=jax.ShapeDtypeStruct((N, OUT), jnp.float32),
    )(*args)


def kernel(x, e, edge_index, W0, b0, g0, beta0, W1, b1, g1, beta1,
           W2, b2, g2, beta2, W3, b3, g3, beta3):
    start = edge_index[0]
    end = edge_index[1]
    pad = E_PAD - E
    gidx = jnp.stack([jnp.pad(start, (0, pad)), jnp.pad(end, (0, pad))])
    gidx = gidx.reshape(2, NSUB, NCHUNK, CHUNK)
    sidx = jnp.stack([jnp.pad(end, (0, pad)), jnp.pad(start, (0, pad))])
    sidx = sidx.reshape(2, NSUB, NCHUNK, CHUNK)
    e_pad = jnp.pad(e, (0, pad)).reshape(NSUB, NCHUNK, CHUNK)
    zeros = jnp.zeros((ROWS_LAST, D), jnp.float32)
    x_flat = jnp.concatenate([x.reshape(-1), jnp.zeros((128,), jnp.float32)])
    mi, mo = _sc_messages(x, x_flat, gidx, sidx, e_pad, zeros)
    return _mlp(mi, mo, x, (W0, W1, W2, W3), (b0, b1, b2, b3),
                (g0, g1, g2, g3), (beta0, beta1, beta2, beta3))


# speedup vs baseline: 1.2106x; 1.1068x over previous
import functools

import jax
import jax.numpy as jnp
from jax import lax
from jax.experimental import pallas as pl
from jax.experimental.pallas import tpu as pltpu
from jax.experimental.pallas import tpu_sc as plsc

N = 10000
E = 320000
D = 128
OUT = 128
EPS = 1e-5

NSUB = 16
CHUNK = 128
NBUF = 3
NCHUNK = 159
E_PAD = NSUB * NCHUNK * CHUNK
EDGES_PER_SUB = E_PAD // NSUB
ROWS_A = 624
ROWS_LAST = N - 15 * ROWS_A
STREAM_ROWS = 64
ROWDMA = CHUNK - STREAM_ROWS


def _sc_messages(x, x_flat, gidx, sidx, e_pad, zeros):
    mesh = plsc.VectorSubcoreMesh(core_axis_name="c", subcore_axis_name="s")

    @functools.partial(
        pl.kernel,
        mesh=mesh,
        out_type=(
            jax.ShapeDtypeStruct((N, D), jnp.float32),
            jax.ShapeDtypeStruct((N, D), jnp.float32),
        ),
        scratch_types=(
            [pltpu.VMEM_SHARED((N, D), jnp.float32)]
            + [pltpu.VMEM((CHUNK, D), jnp.float32) for _ in range(NBUF)]
            + [pltpu.VMEM((CHUNK,), jnp.int32) for _ in range(NBUF)]
            + [pltpu.VMEM((CHUNK,), jnp.int32) for _ in range(NBUF)]
            + [pltpu.VMEM((CHUNK,), jnp.float32) for _ in range(NBUF)]
            + [pltpu.SemaphoreType.DMA for _ in range(3 * NBUF)]
        ),
    )
    def sc_kernel(x_hbm, xf_hbm, gidx_hbm, sidx_hbm, e_hbm, zero_hbm,
                  mi_hbm, mo_hbm, acc, *rest):
        rows = rest[0:NBUF]
        idxg = rest[NBUF:2 * NBUF]
        idxs = rest[2 * NBUF:3 * NBUF]
        ew = rest[3 * NBUF:4 * NBUF]
        gsem = rest[4 * NBUF:5 * NBUF]
        ssem = rest[5 * NBUF:6 * NBUF]
        isem = rest[6 * NBUF:7 * NBUF]
        c = lax.axis_index("c")
        s = lax.axis_index("s")
        row0 = pl.multiple_of(s * ROWS_A, 8)

        def stage_idx(i, b):
            pltpu.async_copy(gidx_hbm.at[c, s, i], idxg[b], isem[b])
            pltpu.async_copy(sidx_hbm.at[c, s, i], idxs[b], isem[b])
            pltpu.async_copy(e_hbm.at[s, i], ew[b], isem[b])

        def wait_idx(b):
            pltpu.make_async_copy(gidx_hbm.at[c, s, 0], idxg[b], isem[b]).wait()
            pltpu.make_async_copy(sidx_hbm.at[c, s, 0], idxs[b], isem[b]).wait()
            pltpu.make_async_copy(e_hbm.at[s, 0], ew[b], isem[b]).wait()

        def issue_gather(b):
            pltpu.async_copy(x_hbm.at[idxg[b].at[pl.ds(0, STREAM_ROWS)]],
                             rows[b].at[pl.ds(0, STREAM_ROWS)], gsem[b])
            for h in range(ROWDMA // 16):
                iv = idxg[b][pl.ds(STREAM_ROWS + h * 16, 16)]
                for j in range(16):
                    r = STREAM_ROWS + h * 16 + j
                    off = pl.multiple_of(iv[j] * D, 8)
                    pltpu.async_copy(xf_hbm.at[pl.ds(off, D)],
                                     rows[b].at[r], gsem[b])

        for b in range(NBUF):
            stage_idx(b, b)
        for b in range(2):
            wait_idx(b)
            issue_gather(b)

        @pl.when(s < NSUB - 1)
        def _():
            pltpu.sync_copy(zero_hbm.at[pl.ds(0, ROWS_A)],
                            acc.at[pl.ds(row0, ROWS_A)])

        @pl.when(s == NSUB - 1)
        def _():
            pltpu.sync_copy(zero_hbm.at[pl.ds(0, ROWS_LAST)],
                            acc.at[pl.ds(row0, ROWS_LAST)])

        plsc.subcore_barrier()

        def group_body(g, carry):
            i0 = g * NBUF
            for b in range(NBUF):
                i = i0 + b
                pltpu.make_async_copy(x_hbm.at[idxg[b]], rows[b],
                                      gsem[b]).wait()

                def grp_body(gg, rcarry, b=b):
                    r0 = pl.multiple_of(gg * 16, 16)
                    ev = ew[b][pl.ds(r0, 16)]
                    sv = idxs[b][pl.ds(r0, 16)]
                    for j in range(16):
                        sc = ev[j]
                        for d in range(D // 16):
                            sl = pl.ds(d * 16, 16)
                            rows[b][r0 + j, sl] = rows[b][r0 + j, sl] * sc
                    pltpu.async_copy(rows[b].at[pl.ds(r0, 16)], acc.at[sv],
                                     ssem[b], add=True)
                    return rcarry

                lax.fori_loop(0, CHUNK // 16, grp_body, 0)

                @pl.when(i >= 1)
                def _(b=b):
                    pltpu.make_async_copy(rows[(b + 2) % NBUF],
                                          acc.at[idxs[(b + 2) % NBUF]],
                                          ssem[(b + 2) % NBUF]).wait()

                @pl.when(i + 2 < NCHUNK)
                def _(b=b):
                    b2 = (b + 2) % NBUF
                    wait_idx(b2)
                    issue_gather(b2)

                @pl.when(i + 3 < NCHUNK)
                def _(b=b, i=i):
                    stage_idx(i + 3, b)
            return carry

        lax.fori_loop(0, NCHUNK // NBUF, group_body, 0)
        pltpu.make_async_copy(rows[(NCHUNK - 1) % NBUF],
                              acc.at[idxs[(NCHUNK - 1) % NBUF]],
                              ssem[(NCHUNK - 1) % NBUF]).wait()
        plsc.subcore_barrier()

        for cid, dst in ((0, mi_hbm), (1, mo_hbm)):
            @pl.when((c == cid) & (s < NSUB - 1))
            def _(dst=dst):
                pltpu.sync_copy(acc.at[pl.ds(row0, ROWS_A)],
                                dst.at[pl.ds(row0, ROWS_A)])

            @pl.when((c == cid) & (s == NSUB - 1))
            def _(dst=dst):
                pltpu.sync_copy(acc.at[pl.ds(row0, ROWS_LAST)],
                                dst.at[pl.ds(row0, ROWS_LAST)])

    return sc_kernel(x, x_flat, gidx, sidx, e_pad, zeros)


def _ln_relu(h, g, b):
    mu = jnp.mean(h, axis=-1, keepdims=True)
    var = jnp.mean((h - mu) ** 2, axis=-1, keepdims=True)
    h = (h - mu) / jnp.sqrt(var + EPS) * g + b
    return jnp.maximum(h, 0.0)


BLK = 1000


def _mlp(mi, mo, x, Ws, bs, gs, betas):
    grid = (N // BLK,)

    def tc_body(mi_ref, mo_ref, x_ref,
                W0_ref, b0_ref, g0_ref, t0_ref,
                W1_ref, b1_ref, g1_ref, t1_ref,
                W2_ref, b2_ref, g2_ref, t2_ref,
                W3_ref, b3_ref, g3_ref, t3_ref,
                out_ref):
        w0 = W0_ref[...]
        h = (
            jnp.dot(mi_ref[...], w0[0:D, :], preferred_element_type=jnp.float32)
            + jnp.dot(mo_ref[...], w0[D:2 * D, :], preferred_element_type=jnp.float32)
            + jnp.dot(x_ref[...], w0[2 * D:3 * D, :], preferred_element_type=jnp.float32)
            + b0_ref[...]
        )
        h = _ln_relu(h, g0_ref[...], t0_ref[...])
        for W_ref, b_ref, g_ref, t_ref in (
            (W1_ref, b1_ref, g1_ref, t1_ref),
            (W2_ref, b2_ref, g2_ref, t2_ref),
            (W3_ref, b3_ref, g3_ref, t3_ref),
        ):
            h = jnp.dot(h, W_ref[...], preferred_element_type=jnp.float32) + b_ref[...]
            h = _ln_relu(h, g_ref[...], t_ref[...])
        out_ref[...] = h

    row_spec = pl.BlockSpec((BLK, D), lambda i: (i, 0))
    vec_spec = pl.BlockSpec((1, OUT), lambda i: (0, 0))
    in_specs = [row_spec, row_spec, row_spec]
    for Wshape in ((3 * D, OUT), (OUT, OUT), (OUT, OUT), (OUT, OUT)):
        in_specs.append(pl.BlockSpec(Wshape, lambda i: (0, 0)))
        in_specs.extend([vec_spec, vec_spec, vec_spec])

    args = [mi, mo, x]
    for i in range(4):
        args.extend([Ws[i], bs[i].reshape(1, OUT), gs[i].reshape(1, OUT),
                     betas[i].reshape(1, OUT)])

    return pl.pallas_call(
        tc_body,
        grid=grid,
        in_specs=in_specs,
        out_specs=pl.BlockSpec((BLK, OUT), lambda i: (i, 0)),
        out_shape=jax.ShapeDtypeStruct((N, OUT), jnp.float32),
    )(*args)


def kernel(x, e, edge_index, W0, b0, g0, beta0, W1, b1, g1, beta1,
           W2, b2, g2, beta2, W3, b3, g3, beta3):
    start = edge_index[0]
    end = edge_index[1]
    pad = E_PAD - E
    gidx = jnp.stack([jnp.pad(start, (0, pad)), jnp.pad(end, (0, pad))])
    gidx = gidx.reshape(2, NSUB, NCHUNK, CHUNK)
    sidx = jnp.stack([jnp.pad(end, (0, pad)), jnp.pad(start, (0, pad))])
    sidx = sidx.reshape(2, NSUB, NCHUNK, CHUNK)
    e_pad = jnp.pad(e, (0, pad)).reshape(NSUB, NCHUNK, CHUNK)
    zeros = jnp.zeros((ROWS_LAST, D), jnp.float32)
    x_flat = jnp.concatenate([x.reshape(-1), jnp.zeros((128,), jnp.float32)])
    mi, mo = _sc_messages(x, x_flat, gidx, sidx, e_pad, zeros)
    return _mlp(mi, mo, x, (W0, W1, W2, W3), (b0, b1, b2, b3),
                (g0, g1, g2, g3), (beta0, beta1, beta2, beta3))
